# baseline (speedup 1.0000x reference)
import functools

import jax
import jax.numpy as jnp
from jax import lax
from jax.experimental import pallas as pl
from jax.experimental.pallas import tpu as pltpu
from jax.experimental.pallas import tpu_sc as plsc

_B, _T, _D = 1024, 200, 128


@functools.cache
def _make_kernel():
    info = plsc.get_sparse_core_info()
    lanes = info.num_lanes
    num_cores = 1
    num_subcores = 8
    num_workers = num_cores * num_subcores
    b_per_w = _B // num_workers
    n_chunk = b_per_w // lanes
    mesh = plsc.VectorSubcoreMesh(
        core_axis_name="c", subcore_axis_name="s", num_cores=num_cores,
        num_subcores=num_subcores,
    )

    @functools.partial(
        pl.kernel,
        mesh=mesh,
        out_type=jax.ShapeDtypeStruct((_B, _D), jnp.float32),
        scratch_types=[
            pltpu.VMEM((b_per_w,), jnp.int32),
            pltpu.VMEM((b_per_w, _D), jnp.float32),
            [pltpu.SemaphoreType.DMA] * n_chunk,
            pltpu.SemaphoreType.DMA,
        ],
    )
    def k(rows_hbm, length_hbm, out_hbm, len_v, gathered_v, gsems, osem):
        wid = lax.axis_index("s") * num_cores + lax.axis_index("c")
        base = wid * b_per_w
        pltpu.sync_copy(length_hbm.at[pl.ds(base, b_per_w)], len_v)
        gathers = []
        for j in range(n_chunk):
            ln = len_v[pl.ds(j * lanes, lanes)]
            idx = (base + j * lanes + lax.iota(jnp.int32, lanes)) * _T + ln - 1
            gathers.append(
                pltpu.async_copy(
                    rows_hbm.at[idx],
                    gathered_v.at[pl.ds(j * lanes, lanes)],
                    gsems[j],
                )
            )
        stores = []
        for j in range(n_chunk):
            gathers[j].wait()
            stores.append(
                pltpu.async_copy(
                    gathered_v.at[pl.ds(j * lanes, lanes)],
                    out_hbm.at[pl.ds(base + j * lanes, lanes)],
                    osem,
                )
            )
        for st in stores:
            st.wait()

    return k


@jax.jit
def kernel(inputs, length):
    rows = inputs.reshape(_B * _T, _D)
    return _make_kernel()(rows, length.astype(jnp.int32))


# speedup vs baseline: 1.0332x; 1.0332x over previous
import functools

import jax
import jax.numpy as jnp
from jax import lax
from jax.experimental import pallas as pl
from jax.experimental.pallas import tpu as pltpu
from jax.experimental.pallas import tpu_sc as plsc

_B, _T, _D = 1024, 200, 128
_CHUNK = 32


@functools.cache
def _make_kernel():
    info = plsc.get_sparse_core_info()
    lanes = info.num_lanes
    num_cores = 1
    num_workers = num_cores * info.num_subcores
    b_per_w = _B // num_workers
    n_chunk = b_per_w // _CHUNK
    mesh = plsc.VectorSubcoreMesh(
        core_axis_name="c", subcore_axis_name="s", num_cores=num_cores
    )

    @functools.partial(
        pl.kernel,
        mesh=mesh,
        out_type=jax.ShapeDtypeStruct((_B, _D), jnp.float32),
        scratch_types=[
            pltpu.VMEM((b_per_w,), jnp.int32),
            pltpu.VMEM((b_per_w,), jnp.int32),
            pltpu.VMEM((b_per_w, _D), jnp.float32),
            [pltpu.SemaphoreType.DMA] * n_chunk,
            pltpu.SemaphoreType.DMA,
        ],
    )
    def k(rows_hbm, length_hbm, out_hbm, len_v, idx_v, gathered_v, gsems, osem):
        wid = lax.axis_index("s") * num_cores + lax.axis_index("c")
        base = wid * b_per_w
        pltpu.sync_copy(length_hbm.at[pl.ds(base, b_per_w)], len_v)
        for j in range(b_per_w // lanes):
            ln = len_v[pl.ds(j * lanes, lanes)]
            b_ids = base + j * lanes + lax.iota(jnp.int32, lanes)
            idx_v[pl.ds(j * lanes, lanes)] = b_ids * _T + ln - 1
        gathers = []
        for j in range(n_chunk):
            gathers.append(
                pltpu.async_copy(
                    rows_hbm.at[idx_v.at[pl.ds(j * _CHUNK, _CHUNK)]],
                    gathered_v.at[pl.ds(j * _CHUNK, _CHUNK)],
                    gsems[j],
                )
            )
        stores = []
        for j in range(n_chunk):
            gathers[j].wait()
            stores.append(
                pltpu.async_copy(
                    gathered_v.at[pl.ds(j * _CHUNK, _CHUNK)],
                    out_hbm.at[pl.ds(base + j * _CHUNK, _CHUNK)],
                    osem,
                )
            )
        for st in stores:
            st.wait()

    return k


@jax.jit
def kernel(inputs, length):
    rows = inputs.reshape(_B * _T, _D)
    return _make_kernel()(rows, length.astype(jnp.int32))
